# baseline (speedup 1.0000x reference)
import functools

import jax
import jax.numpy as jnp
from jax import lax
from jax.experimental import pallas as pl
from jax.experimental.pallas import tpu as pltpu
from jax.experimental.pallas import tpu_sc as plsc

_N = 10000
_E = 320000
_F = 128
_G = 128
_K = 4
_D = 32

_NC = 2
_NS = 16
_NW = _NC * _NS
_EPT = _E // _NW
_CH = 80
_NCHUNK = _EPT // _CH
_NP = 10240
_RPT = _NP // _NS

def _edge_agg_body(h_hbm, eidx_hbm, out_hbm,
                   idxs, rows, acc, isems, gsems):
    cid = lax.axis_index("c")
    sid = lax.axis_index("s")
    wid = cid * _NS + sid

    z16 = jnp.zeros((16,), jnp.float32)

    def zrow(i, carry):
        for j in range(_F // 16):
            rows[0][i, pl.ds(16 * j, 16)] = z16
        return carry

    lax.fori_loop(0, _CH, zrow, 0)
    for j in range(_RPT // _CH):
        pltpu.sync_copy(rows[0], acc.at[pl.ds(sid * _RPT + j * _CH, _CH)])

    def idx_load(c, s):
        b = wid * _EPT + c * _CH
        pltpu.async_copy(eidx_hbm.at[pl.ds(b, _CH)], idxs[s].at[0], isems[s])
        pltpu.async_copy(eidx_hbm.at[pl.ds(_E + b, _CH)], idxs[s].at[1],
                         isems[s])

    def idx_wait(s):
        pltpu.make_async_copy(eidx_hbm.at[pl.ds(0, _CH)],
                              idxs[s].at[0], isems[s]).wait()
        pltpu.make_async_copy(eidx_hbm.at[pl.ds(0, _CH)],
                              idxs[s].at[1], isems[s]).wait()

    def gather(s, r):
        hh = _CH // 2
        pltpu.async_copy(h_hbm.at[idxs[s].at[0, pl.ds(0, hh)]],
                         rows[r].at[pl.ds(0, hh)], gsems[r])
        pltpu.async_copy(h_hbm.at[idxs[s].at[0, pl.ds(hh, hh)]],
                         rows[r].at[pl.ds(hh, hh)], gsems[r])

    def gwait(r):
        pltpu.make_async_copy(h_hbm.at[idxs[0].at[0]], rows[r], gsems[r]).wait()

    def scatter(s, r):
        pltpu.sync_copy(rows[r], acc.at[idxs[s].at[1]], add=True)

    for s in range(8):
        idx_load(s, s)
    plsc.subcore_barrier()
    for s in range(4):
        idx_wait(s)
        gather(s, s)

    def body(g, carry):
        c0 = 8 * g
        for t in range(8):
            r = t % 4
            gwait(r)
            scatter(t, r)

            @pl.when(c0 + t + 4 < _NCHUNK)
            def _():
                idx_wait((t + 4) % 8)
                gather((t + 4) % 8, r)

            @pl.when(c0 + t + 8 < _NCHUNK)
            def _():
                idx_load(c0 + t + 8, t)
        return carry

    lax.fori_loop(0, _NCHUNK // 8, body, 0)
    for t in range(_NCHUNK % 8):
        c = (_NCHUNK // 8) * 8 + t
        r = c % 4
        gwait(r)
        scatter(c % 8, r)
        if c + 4 < _NCHUNK:
            idx_wait((c + 4) % 8)
            gather((c + 4) % 8, r)
    plsc.subcore_barrier()

    pltpu.sync_copy(acc.at[pl.ds(sid * _RPT, _RPT)],
                    out_hbm.at[cid, pl.ds(sid * _RPT, _RPT)])


@functools.cache
def _get_edge_agg():
    mesh = plsc.VectorSubcoreMesh(
        core_axis_name="c", subcore_axis_name="s",
        num_cores=_NC, num_subcores=_NS)
    return functools.partial(
        pl.kernel,
        out_type=jax.ShapeDtypeStruct((_NC, _NP, _F), jnp.float32),
        mesh=mesh,
        scratch_types=[
            [pltpu.VMEM((2, _CH), jnp.int32)] * 8,
            [pltpu.VMEM((_CH, _F), jnp.float32)] * 4,
            pltpu.VMEM_SHARED((_NP, _F), jnp.float32),
            [pltpu.SemaphoreType.DMA] * 8,
            [pltpu.SemaphoreType.DMA] * 4,
        ],
    )(_edge_agg_body)


def _layer_body(h, parts, w1, b1, w2, b2, g, b, o, *, relu_out):
    z = h[...] + parts[0, :_N] + parts[1, :_N]
    t = jnp.maximum(
        jnp.dot(z, w1[...], preferred_element_type=jnp.float32) + b1[...], 0.0)
    t2 = jnp.dot(t, w2[...], preferred_element_type=jnp.float32) + b2[...]
    m = jnp.mean(t2, axis=0, keepdims=True)
    d = t2 - m
    v = jnp.mean(d * d, axis=0, keepdims=True)
    out = d * lax.rsqrt(v + 1e-5) * g[...] + b[...]
    if relu_out:
        out = jnp.maximum(out, 0.0)
    o[...] = out


def _make_layer(relu_out):
    return pl.pallas_call(
        functools.partial(_layer_body, relu_out=relu_out),
        out_shape=jax.ShapeDtypeStruct((_N, _F), jnp.float32),
    )


_layer_relu = _make_layer(True)
_layer_last = _make_layer(False)


def _final_body(h, parts, batch, w1s, b1, w2s, b2, gr, br, o):
    z = h[...] + parts[0, :_N] + parts[1, :_N]
    w1 = jnp.concatenate([w1s[k] for k in range(_K)], axis=1)
    t = jnp.maximum(
        jnp.dot(z, w1, preferred_element_type=jnp.float32) + b1[...], 0.0)
    t2 = jnp.concatenate(
        [jnp.dot(t[:, k * _D:(k + 1) * _D], w2s[k],
                 preferred_element_type=jnp.float32) for k in range(_K)],
        axis=1) + b2[...]
    m = jnp.mean(t2, axis=0, keepdims=True)
    d = t2 - m
    v = jnp.mean(d * d, axis=0, keepdims=True)
    inv = lax.rsqrt(v + 1e-5)
    oh = (batch[...] == lax.broadcasted_iota(jnp.int32, (_N, _G), 1)
          ).astype(jnp.float32)
    praw = lax.dot_general(oh, t2, (((0,), (0,)), ((), ())),
                           preferred_element_type=jnp.float32)
    cnt = lax.dot_general(oh, jnp.ones((_N, 1), jnp.float32),
                          (((0,), (0,)), ((), ())),
                          preferred_element_type=jnp.float32)
    o[...] = (praw - cnt * m) * inv * gr[...] + cnt * br[...]


_final = pl.pallas_call(
    _final_body,
    out_shape=jax.ShapeDtypeStruct((_G, _K * _D), jnp.float32),
)


def kernel(x, edge_index, batch, num_graphs, params):
    del num_graphs
    eidx = edge_index.reshape(2 * _E)

    def row(a):
        return a.reshape(1, -1)

    edge_agg = _get_edge_agg()
    h = x
    for i, lyr in enumerate((_layer_relu, _layer_last)):
        p = params['c%d' % i]
        parts = edge_agg(h, eidx)
        h = lyr(h, parts, p['w1'], row(p['b1']), p['w2'], row(p['b2']),
                row(p['g']), row(p['b']))

    parts = edge_agg(h, eidx)

    def hstack(name):
        return jnp.stack([params['d%d' % k][name] for k in range(_K)])

    def hcat(name):
        return jnp.concatenate(
            [params['d%d' % k][name] for k in range(_K)]).reshape(1, _K * _D)

    pooled = _final(h, parts, batch.reshape(_N, 1), hstack('w1'), hcat('b1'),
                    hstack('w2'), hcat('b2'), hcat('g'), hcat('b'))
    return pooled.reshape(_G, _K, _D)


# speedup vs baseline: 1.0001x; 1.0001x over previous
import functools

import jax
import jax.numpy as jnp
from jax import lax
from jax.experimental import pallas as pl
from jax.experimental.pallas import tpu as pltpu
from jax.experimental.pallas import tpu_sc as plsc

_N = 10000
_E = 320000
_F = 128
_G = 128
_K = 4
_D = 32

_NC = 2
_NS = 16
_NW = _NC * _NS
_EPT = _E // _NW
_CH = 80
_NCHUNK = _EPT // _CH
_NP = 10240
_RPT = _NP // _NS

def _edge_agg_body(h_hbm, eidx_hbm, out_hbm,
                   idxs, rows, acc, isems, gsems):
    cid = lax.axis_index("c")
    sid = lax.axis_index("s")
    wid = cid * _NS + sid

    z16 = jnp.zeros((16,), jnp.float32)

    def zrow(i, carry):
        for j in range(_F // 16):
            rows[0][i, pl.ds(16 * j, 16)] = z16
        return carry

    lax.fori_loop(0, _CH, zrow, 0)
    for j in range(_RPT // _CH):
        pltpu.sync_copy(rows[0], acc.at[pl.ds(sid * _RPT + j * _CH, _CH)])

    def idx_load(c, s):
        b = wid * _EPT + c * _CH
        pltpu.async_copy(eidx_hbm.at[pl.ds(b, _CH)], idxs[s].at[0], isems[s])
        pltpu.async_copy(eidx_hbm.at[pl.ds(_E + b, _CH)], idxs[s].at[1],
                         isems[s])

    def idx_wait(s):
        pltpu.make_async_copy(eidx_hbm.at[pl.ds(0, _CH)],
                              idxs[s].at[0], isems[s]).wait()
        pltpu.make_async_copy(eidx_hbm.at[pl.ds(0, _CH)],
                              idxs[s].at[1], isems[s]).wait()

    def gather(s, r):
        pltpu.async_copy(h_hbm.at[idxs[s].at[0]], rows[r], gsems[r])

    def gwait(r):
        pltpu.make_async_copy(h_hbm.at[idxs[0].at[0]], rows[r], gsems[r]).wait()

    def scatter(s, r):
        pltpu.sync_copy(rows[r], acc.at[idxs[s].at[1]], add=True)

    for s in range(8):
        idx_load(s, s)
    plsc.subcore_barrier()
    for s in range(4):
        idx_wait(s)
        gather(s, s)

    def body(g, carry):
        c0 = 8 * g
        for t in range(8):
            r = t % 4
            gwait(r)
            scatter(t, r)

            @pl.when(c0 + t + 4 < _NCHUNK)
            def _():
                idx_wait((t + 4) % 8)
                gather((t + 4) % 8, r)

            @pl.when(c0 + t + 8 < _NCHUNK)
            def _():
                idx_load(c0 + t + 8, t)
        return carry

    lax.fori_loop(0, _NCHUNK // 8, body, 0)
    for t in range(_NCHUNK % 8):
        c = (_NCHUNK // 8) * 8 + t
        r = c % 4
        gwait(r)
        scatter(c % 8, r)
        if c + 4 < _NCHUNK:
            idx_wait((c + 4) % 8)
            gather((c + 4) % 8, r)
    plsc.subcore_barrier()

    pltpu.sync_copy(acc.at[pl.ds(sid * _RPT, _RPT)],
                    out_hbm.at[cid, pl.ds(sid * _RPT, _RPT)])


@functools.cache
def _get_edge_agg():
    mesh = plsc.VectorSubcoreMesh(
        core_axis_name="c", subcore_axis_name="s",
        num_cores=_NC, num_subcores=_NS)
    return functools.partial(
        pl.kernel,
        out_type=jax.ShapeDtypeStruct((_NC, _NP, _F), jnp.float32),
        mesh=mesh,
        scratch_types=[
            [pltpu.VMEM((2, _CH), jnp.int32)] * 8,
            [pltpu.VMEM((_CH, _F), jnp.float32)] * 4,
            pltpu.VMEM_SHARED((_NP, _F), jnp.float32),
            [pltpu.SemaphoreType.DMA] * 8,
            [pltpu.SemaphoreType.DMA] * 4,
        ],
    )(_edge_agg_body)


def _layer_body(h, parts, w1, b1, w2, b2, g, b, o, *, relu_out):
    z = h[...] + parts[0, :_N] + parts[1, :_N]
    t = jnp.maximum(
        jnp.dot(z, w1[...], preferred_element_type=jnp.float32) + b1[...], 0.0)
    t2 = jnp.dot(t, w2[...], preferred_element_type=jnp.float32) + b2[...]
    m = jnp.mean(t2, axis=0, keepdims=True)
    d = t2 - m
    v = jnp.mean(d * d, axis=0, keepdims=True)
    out = d * lax.rsqrt(v + 1e-5) * g[...] + b[...]
    if relu_out:
        out = jnp.maximum(out, 0.0)
    o[...] = out


def _make_layer(relu_out):
    return pl.pallas_call(
        functools.partial(_layer_body, relu_out=relu_out),
        out_shape=jax.ShapeDtypeStruct((_N, _F), jnp.float32),
    )


_layer_relu = _make_layer(True)
_layer_last = _make_layer(False)


def _final_body(h, parts, batch, w1s, b1, w2s, b2, gr, br, o):
    z = h[...] + parts[0, :_N] + parts[1, :_N]
    w1 = jnp.concatenate([w1s[k] for k in range(_K)], axis=1)
    t = jnp.maximum(
        jnp.dot(z, w1, preferred_element_type=jnp.float32) + b1[...], 0.0)
    t2 = jnp.concatenate(
        [jnp.dot(t[:, k * _D:(k + 1) * _D], w2s[k],
                 preferred_element_type=jnp.float32) for k in range(_K)],
        axis=1) + b2[...]
    m = jnp.mean(t2, axis=0, keepdims=True)
    d = t2 - m
    v = jnp.mean(d * d, axis=0, keepdims=True)
    inv = lax.rsqrt(v + 1e-5)
    oh = (batch[...] == lax.broadcasted_iota(jnp.int32, (_N, _G), 1)
          ).astype(jnp.float32)
    praw = lax.dot_general(oh, t2, (((0,), (0,)), ((), ())),
                           preferred_element_type=jnp.float32)
    cnt = lax.dot_general(oh, jnp.ones((_N, 1), jnp.float32),
                          (((0,), (0,)), ((), ())),
                          preferred_element_type=jnp.float32)
    o[...] = (praw - cnt * m) * inv * gr[...] + cnt * br[...]


_final = pl.pallas_call(
    _final_body,
    out_shape=jax.ShapeDtypeStruct((_G, _K * _D), jnp.float32),
)


def kernel(x, edge_index, batch, num_graphs, params):
    del num_graphs
    eidx = edge_index.reshape(2 * _E)

    def row(a):
        return a.reshape(1, -1)

    edge_agg = _get_edge_agg()
    h = x
    for i, lyr in enumerate((_layer_relu, _layer_last)):
        p = params['c%d' % i]
        parts = edge_agg(h, eidx)
        h = lyr(h, parts, p['w1'], row(p['b1']), p['w2'], row(p['b2']),
                row(p['g']), row(p['b']))

    parts = edge_agg(h, eidx)

    def hstack(name):
        return jnp.stack([params['d%d' % k][name] for k in range(_K)])

    def hcat(name):
        return jnp.concatenate(
            [params['d%d' % k][name] for k in range(_K)]).reshape(1, _K * _D)

    pooled = _final(h, parts, batch.reshape(_N, 1), hstack('w1'), hcat('b1'),
                    hstack('w2'), hcat('b2'), hcat('g'), hcat('b'))
    return pooled.reshape(_G, _K, _D)


# speedup vs baseline: 1.0245x; 1.0244x over previous
import functools

import jax
import jax.numpy as jnp
from jax import lax
from jax.experimental import pallas as pl
from jax.experimental.pallas import tpu as pltpu
from jax.experimental.pallas import tpu_sc as plsc

_N = 10000
_E = 320000
_F = 128
_G = 128
_K = 4
_D = 32

_NC = 2
_NS = 16
_NW = _NC * _NS
_EPT = _E // _NW
_CH = 80
_NCHUNK = _EPT // _CH
_NP = 10240
_RPT = _NP // _NS

def _edge_agg_body(h_hbm, eidx_hbm, out_hbm,
                   idxs, rows, acc, isems, gsems):
    cid = lax.axis_index("c")
    sid = lax.axis_index("s")
    wid = cid * _NS + sid

    z16 = jnp.zeros((16,), jnp.float32)

    def zrow(i, carry):
        for j in range(_F // 16):
            rows[0][i, pl.ds(16 * j, 16)] = z16
        return carry

    lax.fori_loop(0, _CH, zrow, 0)
    for j in range(_RPT // _CH):
        pltpu.sync_copy(rows[0], acc.at[pl.ds(sid * _RPT + j * _CH, _CH)])

    def idx_load(c, s):
        b = wid * _EPT + c * _CH
        pltpu.async_copy(eidx_hbm.at[pl.ds(b, _CH)], idxs[s].at[0], isems[s])
        pltpu.async_copy(eidx_hbm.at[pl.ds(_E + b, _CH)], idxs[s].at[1],
                         isems[s])

    def idx_wait(s):
        pltpu.make_async_copy(eidx_hbm.at[pl.ds(0, _CH)],
                              idxs[s].at[0], isems[s]).wait()
        pltpu.make_async_copy(eidx_hbm.at[pl.ds(0, _CH)],
                              idxs[s].at[1], isems[s]).wait()

    def gather(s, r):
        pltpu.async_copy(h_hbm.at[idxs[s].at[0]], rows[r], gsems[r])

    def gwait(r):
        pltpu.make_async_copy(h_hbm.at[idxs[0].at[0]], rows[r], gsems[r]).wait()

    def scatter(s, r):
        pltpu.sync_copy(rows[r], acc.at[idxs[s].at[1]], add=True)

    for s in range(8):
        idx_load(s, s)
    plsc.subcore_barrier()
    for s in range(4):
        idx_wait(s)
        gather(s, s)

    def body(g, carry):
        c0 = 8 * g
        for t in range(8):
            r = t % 4
            gwait(r)
            scatter(t, r)

            @pl.when(c0 + t + 4 < _NCHUNK)
            def _():
                idx_wait((t + 4) % 8)
                gather((t + 4) % 8, r)

            @pl.when(c0 + t + 8 < _NCHUNK)
            def _():
                idx_load(c0 + t + 8, t)
        return carry

    lax.fori_loop(0, _NCHUNK // 8, body, 0)
    for t in range(_NCHUNK % 8):
        c = (_NCHUNK // 8) * 8 + t
        r = c % 4
        gwait(r)
        scatter(c % 8, r)
        if c + 4 < _NCHUNK:
            idx_wait((c + 4) % 8)
            gather((c + 4) % 8, r)
    plsc.subcore_barrier()

    pltpu.sync_copy(acc.at[pl.ds(sid * _RPT, _RPT)],
                    out_hbm.at[cid, pl.ds(sid * _RPT, _RPT)])


@functools.cache
def _get_edge_agg():
    mesh = plsc.VectorSubcoreMesh(
        core_axis_name="c", subcore_axis_name="s",
        num_cores=_NC, num_subcores=_NS)
    return functools.partial(
        pl.kernel,
        out_type=jax.ShapeDtypeStruct((_NC, _NP, _F), jnp.float32),
        mesh=mesh,
        scratch_types=[
            [pltpu.VMEM((2, _CH), jnp.int32)] * 8,
            [pltpu.VMEM((_CH, _F), jnp.float32)] * 4,
            pltpu.VMEM_SHARED((_NP, _F), jnp.float32),
            [pltpu.SemaphoreType.DMA] * 8,
            [pltpu.SemaphoreType.DMA] * 4,
        ],
    )(_edge_agg_body)


def _layer_body(h, parts, w1, b1, w2, b2, g, b, o, *, relu_out):
    z = h[...] + parts[0, :_N] + parts[1, :_N]
    t = jnp.maximum(
        jnp.dot(z, w1[...], preferred_element_type=jnp.float32) + b1[...], 0.0)
    t2 = jnp.dot(t, w2[...], preferred_element_type=jnp.float32) + b2[...]
    m = jnp.mean(t2, axis=0, keepdims=True)
    d = t2 - m
    v = jnp.mean(d * d, axis=0, keepdims=True)
    out = d * lax.rsqrt(v + 1e-5) * g[...] + b[...]
    if relu_out:
        out = jnp.maximum(out, 0.0)
    o[...] = out


def _make_layer(relu_out):
    return pl.pallas_call(
        functools.partial(_layer_body, relu_out=relu_out),
        out_shape=jax.ShapeDtypeStruct((_N, _F), jnp.float32),
    )


_layer_relu = _make_layer(True)
_layer_last = _make_layer(False)


def _final_body(h, parts, batch, w1, b1, w2, b2, gr, br, o):
    z = h[...] + parts[0, :_N] + parts[1, :_N]
    t = jnp.maximum(
        jnp.dot(z, w1[...], preferred_element_type=jnp.float32) + b1[...], 0.0)
    t2 = jnp.dot(t, w2[...], preferred_element_type=jnp.float32) + b2[...]
    m = jnp.mean(t2, axis=0, keepdims=True)
    d = t2 - m
    v = jnp.mean(d * d, axis=0, keepdims=True)
    inv = lax.rsqrt(v + 1e-5)
    oh = (batch[...] == lax.broadcasted_iota(jnp.int32, (_N, _G), 1)
          ).astype(jnp.float32)
    praw = lax.dot_general(oh, t2, (((0,), (0,)), ((), ())),
                           preferred_element_type=jnp.float32)
    cnt = lax.dot_general(oh, jnp.ones((_N, 1), jnp.float32),
                          (((0,), (0,)), ((), ())),
                          preferred_element_type=jnp.float32)
    o[...] = (praw - cnt * m) * inv * gr[...] + cnt * br[...]


_final = pl.pallas_call(
    _final_body,
    out_shape=jax.ShapeDtypeStruct((_G, _K * _D), jnp.float32),
)


def kernel(x, edge_index, batch, num_graphs, params):
    del num_graphs
    eidx = edge_index.reshape(2 * _E)

    def row(a):
        return a.reshape(1, -1)

    edge_agg = _get_edge_agg()
    h = x
    for i, lyr in enumerate((_layer_relu, _layer_last)):
        p = params['c%d' % i]
        parts = edge_agg(h, eidx)
        h = lyr(h, parts, p['w1'], row(p['b1']), p['w2'], row(p['b2']),
                row(p['g']), row(p['b']))

    parts = edge_agg(h, eidx)

    def hcat(name):
        return jnp.concatenate(
            [params['d%d' % k][name] for k in range(_K)]).reshape(1, _K * _D)

    w1c = jnp.concatenate([params['d%d' % k]['w1'] for k in range(_K)], axis=1)
    w2b = jnp.zeros((_K * _D, _K * _D), jnp.float32)
    for k in range(_K):
        w2b = w2b.at[k * _D:(k + 1) * _D, k * _D:(k + 1) * _D].set(
            params['d%d' % k]['w2'])

    pooled = _final(h, parts, batch.reshape(_N, 1), w1c, hcat('b1'),
                    w2b, hcat('b2'), hcat('g'), hcat('b'))
    return pooled.reshape(_G, _K, _D)
